# baseline (speedup 1.0000x reference)
import numpy as np
import jax
import jax.numpy as jnp
from jax import lax
from jax.experimental import pallas as pl
from jax.experimental.pallas import tpu as pltpu

_K = 512
_BORDER = 8
_R_NMS = 2
_R_G = 3
_GAUSS = np.exp(-2.0 * (np.arange(-_R_G, _R_G + 1) ** 2)).astype(np.float32)
_ONE_BITS = 0x3F800000


def _body(x_ref, out_ref, tmask_ref, topkv_ref):
    h, w = x_ref.shape[1], x_ref.shape[2]
    x = x_ref[0]

    row = lax.broadcasted_iota(jnp.int32, (h, w), 0)
    col = lax.broadcasted_iota(jnp.int32, (h, w), 1)
    inb = (row >= _BORDER) & (row < h - _BORDER) & \
          (col >= _BORDER) & (col < w - _BORDER)
    xt = jnp.where(inb, x, 0.0)

    p1 = jnp.maximum(xt, pltpu.roll(xt, w - 1, 1))
    p3 = jnp.maximum(p1, pltpu.roll(p1, w - 2, 1))
    m1 = jnp.maximum(pltpu.roll(p3, 2, 1),
                     pltpu.roll(xt, w - 2, 1))
    q1 = jnp.maximum(m1, pltpu.roll(m1, h - 1, 0))
    q3 = jnp.maximum(q1, pltpu.roll(q1, h - 2, 0))
    mx = jnp.maximum(pltpu.roll(q3, 2, 0),
                     pltpu.roll(m1, h - 2, 0))
    y = jnp.where(xt >= mx, xt, 0.0)
    topkv_ref[0] = y

    yi = lax.bitcast_convert_type(y, jnp.int32)

    yr = yi.reshape(h // 4, 4, w)
    s0, s1, s2, s3 = yr[:, 0], yr[:, 1], yr[:, 2], yr[:, 3]
    hi01 = jnp.maximum(s0, s1)
    lo01 = jnp.minimum(s0, s1)
    hi23 = jnp.maximum(s2, s3)
    lo23 = jnp.minimum(s2, s3)
    r1 = jnp.maximum(hi01, hi23)
    mid_hi = jnp.minimum(hi01, hi23)
    w01 = hi01 >= hi23
    lo_w = jnp.where(w01, lo01, lo23)
    lo_l = jnp.where(w01, lo23, lo01)
    r2 = jnp.maximum(mid_hi, lo_w)
    r3 = jnp.maximum(jnp.minimum(lo_w, mid_hi), lo_l)
    m3_cap = jnp.max(r3)

    def _cnt_ge(t):
        return lax.cond(
            t > m3_cap,
            lambda: (jnp.sum((r1 >= t).astype(jnp.int32)) +
                     jnp.sum((r2 >= t).astype(jnp.int32))),
            lambda: jnp.sum((yi >= t).astype(jnp.int32)))

    def _bis(_, carry):
        lo, hi, clo, chi = carry
        mid = (lo + hi) // 2
        c = _cnt_ge(mid)
        take = c >= _K
        return (jnp.where(take, mid, lo), jnp.where(take, hi, mid),
                jnp.where(take, c, clo), jnp.where(take, chi, c))

    t_star, _, cnt_ge_star, cnt_gt = lax.fori_loop(
        0, 30, _bis,
        (jnp.int32(0), jnp.int32(_ONE_BITS), jnp.int32(h * w), jnp.int32(0)))

    ties = yi == t_star
    flat = row * w + col

    def _tie_search():
        need = _K - cnt_gt

        def _bis2(_, lo_hi):
            lo, hi = lo_hi
            mid = (lo + hi) // 2
            take = jnp.sum((ties & (flat <= mid)).astype(jnp.int32)) >= need
            return jnp.where(take, lo, mid + 1), jnp.where(take, mid, hi)

        return lax.fori_loop(0, 18, _bis2,
                             (jnp.int32(0), jnp.int32(h * w - 1)))[1]

    m_star = lax.cond(cnt_ge_star == _K,
                      lambda: jnp.int32(h * w - 1), _tie_search)

    tmask = (yi > t_star) | (ties & (flat <= m_star))
    tmask_ref[0] = tmask.astype(jnp.int8)

    z = jnp.where(tmask, y, 0.0)
    t1 = z * _GAUSS[_R_G]
    for d in range(1, _R_G + 1):
        t1 = t1 + _GAUSS[_R_G + d] * (pltpu.roll(z, d, 1) +
                                      pltpu.roll(z, w - d, 1))
    o = t1 * _GAUSS[_R_G]
    for d in range(1, _R_G + 1):
        o = o + _GAUSS[_R_G + d] * (pltpu.roll(t1, d, 0) +
                                    pltpu.roll(t1, h - d, 0))
    out_ref[0] = jnp.clip(o, 0.0, 1.0)


def kernel(im1w_score):
    b, h, w, c = im1w_score.shape
    x = im1w_score.reshape(b, h, w)
    spec = pl.BlockSpec((1, h, w), lambda i: (i, 0, 0))
    out, tmask, topkv = pl.pallas_call(
        _body,
        grid=(b,),
        in_specs=[spec],
        out_specs=[spec, spec, spec],
        out_shape=[
            jax.ShapeDtypeStruct((b, h, w), jnp.float32),
            jax.ShapeDtypeStruct((b, h, w), jnp.int8),
            jax.ShapeDtypeStruct((b, h, w), jnp.float32),
        ],
        compiler_params=pltpu.CompilerParams(
            dimension_semantics=("arbitrary",)),
    )(x)
    return (out.reshape(b, h, w, c),
            tmask.reshape(b, h, w, c).astype(jnp.bool_),
            topkv.reshape(b, h, w, c))


# speedup vs baseline: 2.1783x; 2.1783x over previous
import numpy as np
import jax
import jax.numpy as jnp
from jax import lax
from jax.experimental import pallas as pl
from jax.experimental.pallas import tpu as pltpu

_K = 512
_BORDER = 8
_R_NMS = 2
_R_G = 2
_GAUSS = np.exp(-2.0 * (np.arange(-_R_G, _R_G + 1) ** 2)).astype(np.float32)
_ONE_BITS = 0x3F800000


def _body(x_ref, out_ref, tmask_ref, topkv_ref):
    h, w = x_ref.shape[1], x_ref.shape[2]
    x = x_ref[0]

    row = lax.broadcasted_iota(jnp.int32, (h, w), 0)
    col = lax.broadcasted_iota(jnp.int32, (h, w), 1)
    span = jnp.uint32(h - 2 * _BORDER)
    inb = ((row - _BORDER).astype(jnp.uint32) < span) & \
          ((col - _BORDER).astype(jnp.uint32) < span)
    xt = jnp.where(inb, x, 0.0)

    p1 = jnp.maximum(xt, pltpu.roll(xt, w - 1, 1))
    p3 = jnp.maximum(p1, pltpu.roll(p1, w - 2, 1))
    m1 = jnp.maximum(pltpu.roll(p3, 2, 1),
                     pltpu.roll(xt, w - 2, 1))
    q1 = jnp.maximum(m1, pltpu.roll(m1, h - 1, 0))
    q3 = jnp.maximum(q1, pltpu.roll(q1, h - 2, 0))
    mx = jnp.maximum(pltpu.roll(q3, 2, 0),
                     pltpu.roll(m1, h - 2, 0))
    y = jnp.where(xt >= mx, xt, 0.0)
    topkv_ref[0] = y

    yi = lax.bitcast_convert_type(y, jnp.int32)

    def _cnt_ge(t):
        return jnp.sum((yi >= t).astype(jnp.int32))

    def _bis(_, carry):
        lo, hi, clo, chi = carry
        mid = (lo + hi) // 2
        c = _cnt_ge(mid)
        take = c >= _K
        return (jnp.where(take, mid, lo), jnp.where(take, hi, mid),
                jnp.where(take, c, clo), jnp.where(take, chi, c))

    t_star, _, cnt_ge_star, cnt_gt = lax.fori_loop(
        0, 30, _bis,
        (jnp.int32(0), jnp.int32(_ONE_BITS), jnp.int32(h * w), jnp.int32(0)))

    ties = yi == t_star
    flat = row * w + col

    def _tie_search():
        need = _K - cnt_gt

        def _bis2(_, lo_hi):
            lo, hi = lo_hi
            mid = (lo + hi) // 2
            take = jnp.sum((ties & (flat <= mid)).astype(jnp.int32)) >= need
            return jnp.where(take, lo, mid + 1), jnp.where(take, mid, hi)

        return lax.fori_loop(0, 18, _bis2,
                             (jnp.int32(0), jnp.int32(h * w - 1)))[1]

    m_star = lax.cond(cnt_ge_star == _K,
                      lambda: jnp.int32(h * w - 1), _tie_search)

    tmask = (yi > t_star) | (ties & (flat <= m_star))
    tmask_ref[0] = tmask.astype(jnp.int8)

    z = jnp.where(tmask, y, 0.0)
    t1 = z * _GAUSS[_R_G]
    for d in range(1, _R_G + 1):
        t1 = t1 + _GAUSS[_R_G + d] * (pltpu.roll(z, d, 1) +
                                      pltpu.roll(z, w - d, 1))
    o = t1 * _GAUSS[_R_G]
    for d in range(1, _R_G + 1):
        o = o + _GAUSS[_R_G + d] * (pltpu.roll(t1, d, 0) +
                                    pltpu.roll(t1, h - d, 0))
    out_ref[0] = jnp.clip(o, 0.0, 1.0)


def kernel(im1w_score):
    b, h, w, c = im1w_score.shape
    x = im1w_score.reshape(b, h, w)
    spec = pl.BlockSpec((1, h, w), lambda i: (i, 0, 0))
    out, tmask, topkv = pl.pallas_call(
        _body,
        grid=(b,),
        in_specs=[spec],
        out_specs=[spec, spec, spec],
        out_shape=[
            jax.ShapeDtypeStruct((b, h, w), jnp.float32),
            jax.ShapeDtypeStruct((b, h, w), jnp.int8),
            jax.ShapeDtypeStruct((b, h, w), jnp.float32),
        ],
        compiler_params=pltpu.CompilerParams(
            dimension_semantics=("arbitrary",)),
    )(x)
    return (out.reshape(b, h, w, c),
            tmask.reshape(b, h, w, c).astype(jnp.bool_),
            topkv.reshape(b, h, w, c))


# speedup vs baseline: 2.5047x; 1.1498x over previous
import numpy as np
import jax
import jax.numpy as jnp
from jax import lax
from jax.experimental import pallas as pl
from jax.experimental.pallas import tpu as pltpu

_K = 512
_BORDER = 8
_R_G = 2
_GAUSS = np.exp(-2.0 * (np.arange(-_R_G, _R_G + 1) ** 2)).astype(np.float32)
_ONE_BITS = 0x3F800000
_IMGS = 2


def _nms_survivors(x, h, w):
    row = lax.broadcasted_iota(jnp.int32, (h, w), 0)
    col = lax.broadcasted_iota(jnp.int32, (h, w), 1)
    span = jnp.uint32(h - 2 * _BORDER)
    inb = ((row - _BORDER).astype(jnp.uint32) < span) & \
          ((col - _BORDER).astype(jnp.uint32) < span)
    xt = jnp.where(inb, x, 0.0)

    p1 = jnp.maximum(xt, pltpu.roll(xt, w - 1, 1))
    p3 = jnp.maximum(p1, pltpu.roll(p1, w - 2, 1))
    m1 = jnp.maximum(pltpu.roll(p3, 2, 1),
                     pltpu.roll(xt, w - 2, 1))
    q1 = jnp.maximum(m1, pltpu.roll(m1, h - 1, 0))
    q3 = jnp.maximum(q1, pltpu.roll(q1, h - 2, 0))
    mx = jnp.maximum(pltpu.roll(q3, 2, 0),
                     pltpu.roll(m1, h - 2, 0))
    return jnp.where(xt >= mx, xt, 0.0)


def _topk_mask(yi, cnt_ge_star, cnt_gt, t_star, h, w):
    ties = yi == t_star
    row = lax.broadcasted_iota(jnp.int32, (h, w), 0)
    col = lax.broadcasted_iota(jnp.int32, (h, w), 1)
    flat = row * w + col

    def _tie_search():
        need = _K - cnt_gt

        def _bis2(_, lo_hi):
            lo, hi = lo_hi
            mid = (lo + hi) // 2
            take = jnp.sum((ties & (flat <= mid)).astype(jnp.int32)) >= need
            return jnp.where(take, lo, mid + 1), jnp.where(take, mid, hi)

        return lax.fori_loop(0, 18, _bis2,
                             (jnp.int32(0), jnp.int32(h * w - 1)))[1]

    m_star = lax.cond(cnt_ge_star == _K,
                      lambda: jnp.int32(h * w - 1), _tie_search)
    return (yi > t_star) | (ties & (flat <= m_star))


def _gauss5(z, h, w):
    t1 = z * _GAUSS[_R_G]
    for d in range(1, _R_G + 1):
        t1 = t1 + _GAUSS[_R_G + d] * (pltpu.roll(z, d, 1) +
                                      pltpu.roll(z, w - d, 1))
    o = t1 * _GAUSS[_R_G]
    for d in range(1, _R_G + 1):
        o = o + _GAUSS[_R_G + d] * (pltpu.roll(t1, d, 0) +
                                    pltpu.roll(t1, h - d, 0))
    return jnp.clip(o, 0.0, 1.0)


def _body(x_ref, out_ref, tmask_ref, topkv_ref):
    h, w = x_ref.shape[1], x_ref.shape[2]

    yis = []
    for i in range(_IMGS):
        y = _nms_survivors(x_ref[i], h, w)
        topkv_ref[i] = y
        yis.append(lax.bitcast_convert_type(y, jnp.int32))

    def _bis(_, carry):
        nxt = []
        for i in range(_IMGS):
            lo, hi, clo, chi = carry[i]
            mid = (lo + hi) // 2
            c = jnp.sum((yis[i] >= mid).astype(jnp.int32))
            take = c >= _K
            nxt.append((jnp.where(take, mid, lo), jnp.where(take, hi, mid),
                        jnp.where(take, c, clo), jnp.where(take, chi, c)))
        return tuple(nxt)

    init = tuple((jnp.int32(0), jnp.int32(_ONE_BITS),
                  jnp.int32(h * w), jnp.int32(0)) for _ in range(_IMGS))
    res = lax.fori_loop(0, 30, _bis, init)

    for i in range(_IMGS):
        t_star, _, cnt_ge_star, cnt_gt = res[i]
        tmask = _topk_mask(yis[i], cnt_ge_star, cnt_gt, t_star, h, w)
        tmask_ref[i] = tmask.astype(jnp.int8)
        y = lax.bitcast_convert_type(yis[i], jnp.float32)
        out_ref[i] = _gauss5(jnp.where(tmask, y, 0.0), h, w)


def kernel(im1w_score):
    b, h, w, c = im1w_score.shape
    x = im1w_score.reshape(b, h, w)
    spec = pl.BlockSpec((_IMGS, h, w), lambda i: (i, 0, 0))
    out, tmask, topkv = pl.pallas_call(
        _body,
        grid=(b // _IMGS,),
        in_specs=[spec],
        out_specs=[spec, spec, spec],
        out_shape=[
            jax.ShapeDtypeStruct((b, h, w), jnp.float32),
            jax.ShapeDtypeStruct((b, h, w), jnp.int8),
            jax.ShapeDtypeStruct((b, h, w), jnp.float32),
        ],
        compiler_params=pltpu.CompilerParams(
            dimension_semantics=("arbitrary",)),
    )(x)
    return (out.reshape(b, h, w, c),
            tmask.reshape(b, h, w, c).astype(jnp.bool_),
            topkv.reshape(b, h, w, c))


# speedup vs baseline: 2.6943x; 1.0757x over previous
import numpy as np
import jax
import jax.numpy as jnp
from jax import lax
from jax.experimental import pallas as pl
from jax.experimental.pallas import tpu as pltpu

_K = 512
_BORDER = 8
_R_G = 2
_GAUSS = np.exp(-2.0 * (np.arange(-_R_G, _R_G + 1) ** 2)).astype(np.float32)
_ONE_BITS = 0x3F800000
_IMGS = 4


def _nms_survivors(x, h, w):
    row = lax.broadcasted_iota(jnp.int32, (h, w), 0)
    col = lax.broadcasted_iota(jnp.int32, (h, w), 1)
    span = jnp.uint32(h - 2 * _BORDER)
    inb = ((row - _BORDER).astype(jnp.uint32) < span) & \
          ((col - _BORDER).astype(jnp.uint32) < span)
    xt = jnp.where(inb, x, 0.0)

    p1 = jnp.maximum(xt, pltpu.roll(xt, w - 1, 1))
    p3 = jnp.maximum(p1, pltpu.roll(p1, w - 2, 1))
    m1 = jnp.maximum(pltpu.roll(p3, 2, 1),
                     pltpu.roll(xt, w - 2, 1))
    q1 = jnp.maximum(m1, pltpu.roll(m1, h - 1, 0))
    q3 = jnp.maximum(q1, pltpu.roll(q1, h - 2, 0))
    mx = jnp.maximum(pltpu.roll(q3, 2, 0),
                     pltpu.roll(m1, h - 2, 0))
    return jnp.where(xt >= mx, xt, 0.0)


def _topk_mask(yi, cnt_ge_star, cnt_gt, t_star, h, w):
    ties = yi == t_star
    row = lax.broadcasted_iota(jnp.int32, (h, w), 0)
    col = lax.broadcasted_iota(jnp.int32, (h, w), 1)
    flat = row * w + col

    def _tie_search():
        need = _K - cnt_gt

        def _bis2(_, lo_hi):
            lo, hi = lo_hi
            mid = (lo + hi) // 2
            take = jnp.sum((ties & (flat <= mid)).astype(jnp.int32)) >= need
            return jnp.where(take, lo, mid + 1), jnp.where(take, mid, hi)

        return lax.fori_loop(0, 18, _bis2,
                             (jnp.int32(0), jnp.int32(h * w - 1)))[1]

    m_star = lax.cond(cnt_ge_star == _K,
                      lambda: jnp.int32(h * w - 1), _tie_search)
    return (yi > t_star) | (ties & (flat <= m_star))


def _gauss5(z, h, w):
    t1 = z * _GAUSS[_R_G]
    for d in range(1, _R_G + 1):
        t1 = t1 + _GAUSS[_R_G + d] * (pltpu.roll(z, d, 1) +
                                      pltpu.roll(z, w - d, 1))
    o = t1 * _GAUSS[_R_G]
    for d in range(1, _R_G + 1):
        o = o + _GAUSS[_R_G + d] * (pltpu.roll(t1, d, 0) +
                                    pltpu.roll(t1, h - d, 0))
    return jnp.clip(o, 0.0, 1.0)


def _body(x_ref, out_ref, tmask_ref, topkv_ref):
    h, w = x_ref.shape[1], x_ref.shape[2]

    yis = []
    for i in range(_IMGS):
        y = _nms_survivors(x_ref[i], h, w)
        topkv_ref[i] = y
        yis.append(lax.bitcast_convert_type(y, jnp.int32))

    def _bis(_, carry):
        nxt = []
        for i in range(_IMGS):
            lo, hi, clo, chi = carry[i]
            mid = (lo + hi) // 2
            c = jnp.sum((yis[i] >= mid).astype(jnp.int32))
            take = c >= _K
            nxt.append((jnp.where(take, mid, lo), jnp.where(take, hi, mid),
                        jnp.where(take, c, clo), jnp.where(take, chi, c)))
        return tuple(nxt)

    init = tuple((jnp.int32(0), jnp.int32(_ONE_BITS),
                  jnp.int32(h * w), jnp.int32(0)) for _ in range(_IMGS))
    res = lax.fori_loop(0, 30, _bis, init)

    for i in range(_IMGS):
        t_star, _, cnt_ge_star, cnt_gt = res[i]
        tmask = _topk_mask(yis[i], cnt_ge_star, cnt_gt, t_star, h, w)
        tmask_ref[i] = tmask.astype(jnp.int8)
        y = lax.bitcast_convert_type(yis[i], jnp.float32)
        out_ref[i] = _gauss5(jnp.where(tmask, y, 0.0), h, w)


def kernel(im1w_score):
    b, h, w, c = im1w_score.shape
    x = im1w_score.reshape(b, h, w)
    spec = pl.BlockSpec((_IMGS, h, w), lambda i: (i, 0, 0))
    out, tmask, topkv = pl.pallas_call(
        _body,
        grid=(b // _IMGS,),
        in_specs=[spec],
        out_specs=[spec, spec, spec],
        out_shape=[
            jax.ShapeDtypeStruct((b, h, w), jnp.float32),
            jax.ShapeDtypeStruct((b, h, w), jnp.int8),
            jax.ShapeDtypeStruct((b, h, w), jnp.float32),
        ],
        compiler_params=pltpu.CompilerParams(
            dimension_semantics=("arbitrary",)),
    )(x)
    return (out.reshape(b, h, w, c),
            tmask.reshape(b, h, w, c).astype(jnp.bool_),
            topkv.reshape(b, h, w, c))
